# Initial kernel scaffold; baseline (speedup 1.0000x reference)
import jax
import jax.numpy as jnp
from jax.experimental import pallas as pl


def kernel(x):
    raise NotImplementedError("write your pallas kernel here")


# speedup vs baseline: 1.0610x; 1.0610x over previous
import functools

import jax
import jax.numpy as jnp
from jax import lax
from jax.experimental import pallas as pl
from jax.experimental.pallas import tpu as pltpu
from jax.experimental.pallas import tpu_sc as plsc

_PERIOD = 4
_START = 1
_NC = 2
_NS = 16
_NW = _NC * _NS

_CHUNK = 8192


def _decimate_body(x_hbm, o_hbm, in_v0, in_v1, out_v0, out_v1,
                   sem_in0, sem_in1, sem_out0, sem_out1, *, n_out):
    wid = lax.axis_index("s") * _NC + lax.axis_index("c")
    per_w = n_out // _NW
    base_out = wid * per_w
    n_chunks = per_w // _CHUNK
    idx0 = lax.iota(jnp.int32, 16) * _PERIOD + _START

    in_bufs = (in_v0, in_v1)
    out_bufs = (out_v0, out_v1)
    sem_ins = (sem_in0, sem_in1)
    sem_outs = (sem_out0, sem_out1)

    def in_copy(c):
        off = (base_out + c * _CHUNK) * _PERIOD
        return pltpu.async_copy(
            x_hbm.at[pl.ds(off, _CHUNK * _PERIOD)], in_bufs[c % 2],
            sem_ins[c % 2])

    def out_copy(c):
        off = base_out + c * _CHUNK
        return pltpu.async_copy(
            out_bufs[c % 2], o_hbm.at[pl.ds(off, _CHUNK)], sem_outs[c % 2])

    in_copy(0)
    for c in range(n_chunks):
        b = c % 2
        pltpu.make_async_copy(
            x_hbm.at[pl.ds(0, _CHUNK * _PERIOD)], in_bufs[b],
            sem_ins[b]).wait()
        if c + 1 < n_chunks:
            in_copy(c + 1)
        if c >= 2:
            pltpu.make_async_copy(
                out_bufs[b], o_hbm.at[pl.ds(0, _CHUNK)], sem_outs[b]).wait()

        def vec_body(i, _, b=b):
            out_bufs[b][pl.ds(i * 16, 16)] = plsc.load_gather(
                in_bufs[b], [idx0 + i * (16 * _PERIOD)])
            return 0

        lax.fori_loop(0, _CHUNK // 16, vec_body, 0, unroll=8)
        out_copy(c)
    for c in (n_chunks - 2, n_chunks - 1):
        pltpu.make_async_copy(
            out_bufs[c % 2], o_hbm.at[pl.ds(0, _CHUNK)],
            sem_outs[c % 2]).wait()


@functools.partial(jax.jit, static_argnums=(1,))
def _decimate_flat(x_flat, n_out):
    body = functools.partial(_decimate_body, n_out=n_out)
    return pl.kernel(
        body,
        out_type=jax.ShapeDtypeStruct((n_out,), jnp.float32),
        mesh=plsc.VectorSubcoreMesh(core_axis_name="c", subcore_axis_name="s"),
        scratch_types=[
            pltpu.VMEM((_CHUNK * _PERIOD,), jnp.float32),
            pltpu.VMEM((_CHUNK * _PERIOD,), jnp.float32),
            pltpu.VMEM((_CHUNK,), jnp.float32),
            pltpu.VMEM((_CHUNK,), jnp.float32),
            pltpu.SemaphoreType.DMA,
            pltpu.SemaphoreType.DMA,
            pltpu.SemaphoreType.DMA,
            pltpu.SemaphoreType.DMA,
        ],
        compiler_params=pltpu.CompilerParams(needs_layout_passes=False),
    )(x_flat)


def kernel(x):
    shape = x.shape
    t = shape[-1]
    assert t % _PERIOD == 0
    n_out_t = t // _PERIOD
    n_out = x.size // _PERIOD
    y = _decimate_flat(x.reshape(-1), n_out)
    return y.reshape(*shape[:-1], n_out_t)
